# baseline (speedup 1.0000x reference)
import jax
import jax.numpy as jnp
from jax.experimental import pallas as pl

N = 16384
D = 256
M = 8
NCODES = 256
DSUB = D // M
BN = 2048


def _pdt_body(xb_ref, cbt_ref, out_ref):
    xb = xb_ref[:]
    xb16 = xb.astype(jnp.bfloat16)
    xnorm = jnp.sum(xb * xb, axis=-1)
    sum_d = jnp.zeros((BN,), jnp.float32)
    sum_dot = jnp.zeros((BN,), jnp.float32)
    for m in range(M):
        xc = xb[:, m * DSUB:(m + 1) * DSUB]
        xc16 = xb16[:, m * DSUB:(m + 1) * DSUB]
        cbt = cbt_ref[m]
        cbn = jnp.sum(cbt * cbt, axis=0)
        s = cbn[None, :] - 2.0 * jnp.dot(xc16, cbt.astype(jnp.bfloat16),
                                         preferred_element_type=jnp.float32)
        smin = jnp.min(s, axis=-1)
        mask = s == smin[:, None]
        cbn_sel = jnp.max(jnp.where(mask, cbn[None, :], -jnp.inf), axis=-1)
        xcn = jnp.sum(xc * xc, axis=-1)
        sum_d = sum_d + jnp.maximum(xcn + smin, 0.0)
        sum_dot = sum_dot + 0.5 * (cbn_sel - smin)
    out_ref[:] = jnp.sqrt(sum_d) + jnp.abs(sum_dot - xnorm)


def kernel(x, codebook):
    cbt = jnp.transpose(codebook, (0, 2, 1))
    return pl.pallas_call(
        _pdt_body,
        grid=(N // BN,),
        in_specs=[
            pl.BlockSpec((BN, D), lambda i: (i, 0)),
            pl.BlockSpec((M, DSUB, NCODES), lambda i: (0, 0, 0)),
        ],
        out_specs=pl.BlockSpec((BN,), lambda i: (i,)),
        out_shape=jax.ShapeDtypeStruct((N,), jnp.float32),
    )(x, cbt)


# speedup vs baseline: 1.1211x; 1.1211x over previous
import jax
import jax.numpy as jnp
from jax.experimental import pallas as pl

N = 16384
D = 256
M = 8
NCODES = 256
DSUB = D // M
BN = 2048
KP = 40

_NEG_BIG = -3.0e38


def _pdt_body(xr_ref, w_ref, cbn_ref, out_ref):
    sum_s = jnp.zeros((BN,), jnp.float32)
    sum_dot = jnp.zeros((BN,), jnp.float32)
    xnorm = jnp.zeros((BN,), jnp.float32)
    for m in range(M):
        xrm = xr_ref[m * KP:(m + 1) * KP]
        xm = xrm[:DSUB].astype(jnp.float32)
        xnorm = xnorm + jnp.sum(xm * xm, axis=0)
        s = jnp.dot(w_ref[m], xrm,
                    preferred_element_type=jnp.float32)
        smin = jnp.min(s, axis=0)
        mask = s == smin[None, :]
        cbn_col = cbn_ref[m][:, None]
        cbn_sel = jnp.max(jnp.where(mask, cbn_col, _NEG_BIG), axis=0)
        sum_s = sum_s + smin
        sum_dot = sum_dot + 0.5 * (cbn_sel - smin)
    sum_d = jnp.maximum(xnorm + sum_s, 0.0)
    out_ref[:] = jnp.sqrt(sum_d) + jnp.abs(sum_dot - xnorm)


def kernel(x, codebook):
    xt = x.T.astype(jnp.bfloat16)
    ones = jnp.ones((1, N), jnp.bfloat16)
    zpad = jnp.zeros((KP - DSUB - 1, N), jnp.bfloat16)
    xr = jnp.concatenate(
        [jnp.concatenate([xt[m * DSUB:(m + 1) * DSUB], ones, zpad], axis=0)
         for m in range(M)], axis=0)
    cbn = jnp.sum(codebook * codebook, axis=-1)
    w = jnp.concatenate(
        [-2.0 * codebook, cbn[:, :, None],
         jnp.zeros((M, NCODES, KP - DSUB - 1), jnp.float32)],
        axis=-1).astype(jnp.bfloat16)
    return pl.pallas_call(
        _pdt_body,
        grid=(N // BN,),
        in_specs=[
            pl.BlockSpec((M * KP, BN), lambda i: (0, i)),
            pl.BlockSpec((M, NCODES, KP), lambda i: (0, 0, 0)),
            pl.BlockSpec((M, NCODES), lambda i: (0, 0)),
        ],
        out_specs=pl.BlockSpec((BN,), lambda i: (i,)),
        out_shape=jax.ShapeDtypeStruct((N,), jnp.float32),
    )(xr, w, cbn)


# speedup vs baseline: 2.2633x; 2.0187x over previous
import jax
import jax.numpy as jnp
from jax import lax
from jax.experimental import pallas as pl

N = 16384
D = 256
M = 8
NCODES = 256
DSUB = D // M
BN = 2048

_NEG_BIG = -3.0e38
_DN = (((1,), (1,)), ((), ()))


def _pdt_body(x_ref, w_ref, cbn_ref, out_ref):
    xb = x_ref[:]
    xb16 = xb.astype(jnp.bfloat16)
    xsq16 = (xb * xb).astype(jnp.bfloat16)
    ones8 = jnp.ones((8, D), jnp.bfloat16)
    xnorm = lax.dot_general(ones8, xsq16, _DN,
                            preferred_element_type=jnp.float32)[0]
    sum_s = jnp.zeros((BN,), jnp.float32)
    sum_dot = jnp.zeros((BN,), jnp.float32)
    for m in range(M):
        xc16 = xb16[:, m * DSUB:(m + 1) * DSUB]
        cbn_col = cbn_ref[m][:, None]
        s = cbn_col + lax.dot_general(
            w_ref[m], xc16, _DN,
            preferred_element_type=jnp.float32)
        smin = jnp.min(s, axis=0)
        mask = s == smin[None, :]
        cbn_sel = jnp.max(jnp.where(mask, cbn_col, _NEG_BIG), axis=0)
        sum_s = sum_s + smin
        sum_dot = sum_dot + 0.5 * (cbn_sel - smin)
    sum_d = jnp.maximum(xnorm + sum_s, 0.0)
    out_ref[:] = jnp.sqrt(sum_d) + jnp.abs(sum_dot - xnorm)


def kernel(x, codebook):
    cbn = jnp.sum(codebook * codebook, axis=-1)
    w = (-2.0 * codebook).astype(jnp.bfloat16)
    return pl.pallas_call(
        _pdt_body,
        grid=(N // BN,),
        in_specs=[
            pl.BlockSpec((BN, D), lambda i: (i, 0)),
            pl.BlockSpec((M, NCODES, DSUB), lambda i: (0, 0, 0)),
            pl.BlockSpec((M, NCODES), lambda i: (0, 0)),
        ],
        out_specs=pl.BlockSpec((BN,), lambda i: (i,)),
        out_shape=jax.ShapeDtypeStruct((N,), jnp.float32),
    )(x, w, cbn)


# speedup vs baseline: 2.6948x; 1.1906x over previous
import jax
import jax.numpy as jnp
from jax import lax
from jax.experimental import pallas as pl

N = 16384
D = 256
M = 8
NCODES = 256
DSUB = D // M
BN = 2048

_NEG_BIG = -3.0e38
_DN = (((1,), (1,)), ((), ()))


def _pdt_body(x_ref, w_ref, cbn_ref, out_ref):
    xb = x_ref[:]
    xb16 = xb.astype(jnp.bfloat16)
    xsq16 = xb16 * xb16
    ones8 = jnp.ones((8, D), jnp.bfloat16)
    xnorm = lax.dot_general(ones8, xsq16, _DN,
                            preferred_element_type=jnp.float32)[0]
    sum_s = jnp.zeros((BN,), jnp.float32)
    sum_dot = jnp.zeros((BN,), jnp.float32)
    neg_big16 = jnp.bfloat16(-3.0e38)
    for m in range(M):
        xc16 = xb16[:, m * DSUB:(m + 1) * DSUB]
        cbn_col = cbn_ref[m][:, None]
        s = (cbn_col.astype(jnp.float32) + lax.dot_general(
            w_ref[m], xc16, _DN,
            preferred_element_type=jnp.float32)
             ).astype(jnp.bfloat16)
        smin = jnp.min(s, axis=0)
        mask = s == smin[None, :]
        cbn_sel = jnp.max(jnp.where(mask, cbn_col, neg_big16), axis=0)
        sum_s = sum_s + smin.astype(jnp.float32)
        sum_dot = sum_dot + 0.5 * (cbn_sel.astype(jnp.float32)
                                   - smin.astype(jnp.float32))
    sum_d = jnp.maximum(xnorm + sum_s, 0.0)
    out_ref[:] = jnp.sqrt(sum_d) + jnp.abs(sum_dot - xnorm)


def kernel(x, codebook):
    cbn = jnp.sum(codebook * codebook,
                  axis=-1).astype(jnp.bfloat16)
    w = (-2.0 * codebook).astype(jnp.bfloat16)
    return pl.pallas_call(
        _pdt_body,
        grid=(N // BN,),
        in_specs=[
            pl.BlockSpec((BN, D), lambda i: (i, 0)),
            pl.BlockSpec((M, NCODES, DSUB), lambda i: (0, 0, 0)),
            pl.BlockSpec((M, NCODES), lambda i: (0, 0)),
        ],
        out_specs=pl.BlockSpec((BN,), lambda i: (i,)),
        out_shape=jax.ShapeDtypeStruct((N,), jnp.float32),
    )(x, w, cbn)


# speedup vs baseline: 2.9909x; 1.1099x over previous
import jax
import jax.numpy as jnp
from jax import lax
from jax.experimental import pallas as pl

N = 16384
D = 256
M = 8
NCODES = 256
DSUB = D // M
BN = 4096

_NEG_BIG = -3.0e38
_DN = (((1,), (1,)), ((), ()))


def _pdt_body(x_ref, w_ref, cbn_ref, out_ref):
    xb = x_ref[:]
    xb16 = xb.astype(jnp.bfloat16)
    xsq16 = xb16 * xb16
    ones8 = jnp.ones((8, D), jnp.bfloat16)
    xnorm = lax.dot_general(ones8, xsq16, _DN,
                            preferred_element_type=jnp.float32)[0]
    sum_s = jnp.zeros((BN,), jnp.float32)
    sum_dot = jnp.zeros((BN,), jnp.float32)
    neg_big16 = jnp.bfloat16(-3.0e38)
    for m in range(M):
        xc16 = xb16[:, m * DSUB:(m + 1) * DSUB]
        cbn_col = cbn_ref[m][:, None]
        s = cbn_col + lax.dot_general(
            w_ref[m], xc16, _DN,
            preferred_element_type=jnp.float32,
        ).astype(jnp.bfloat16)
        smin = jnp.min(s, axis=0)
        mask = s == smin[None, :]
        cbn_sel = jnp.max(jnp.where(mask, cbn_col, neg_big16), axis=0)
        sum_s = sum_s + smin.astype(jnp.float32)
        sum_dot = sum_dot + 0.5 * (cbn_sel.astype(jnp.float32)
                                   - smin.astype(jnp.float32))
    sum_d = jnp.maximum(xnorm + sum_s, 0.0)
    out_ref[:] = jnp.sqrt(sum_d) + jnp.abs(sum_dot - xnorm)


def kernel(x, codebook):
    cbn = jnp.sum(codebook * codebook,
                  axis=-1).astype(jnp.bfloat16)
    w = (-2.0 * codebook).astype(jnp.bfloat16)
    return pl.pallas_call(
        _pdt_body,
        grid=(N // BN,),
        in_specs=[
            pl.BlockSpec((BN, D), lambda i: (i, 0)),
            pl.BlockSpec((M, NCODES, DSUB), lambda i: (0, 0, 0)),
            pl.BlockSpec((M, NCODES), lambda i: (0, 0)),
        ],
        out_specs=pl.BlockSpec((BN,), lambda i: (i,)),
        out_shape=jax.ShapeDtypeStruct((N,), jnp.float32),
    )(x, w, cbn)
